# baseline (speedup 1.0000x reference)
import numpy as np
import jax
import jax.numpy as jnp
from jax import lax
from jax.experimental import pallas as pl
from jax.experimental.pallas import tpu as pltpu
from jax.experimental.pallas import tpu_sc as plsc

_H = 768
_M2D = 1024
_SUB = 96
_B = 128
_S = 512
_MAXP = 512

_NPOS = 3 * _M2D
_NDIST = 5 * (2 * _M2D + 1)
_NBR = _NPOS + _NDIST

_NW = 32
_SPW = _S // _NW
_CH = 8
_RPC = _CH * 16
_BG = _B // _CH
_NCHUNK = _SPW * _BG


def _pe_table():
    position = np.arange(_MAXP)[:, None].astype(np.float32)
    div_term = np.exp(
        np.arange(0, _H, 2).astype(np.float32) * (-np.log(10000.0) / _H))
    pe = np.zeros((_MAXP, _H), dtype=np.float32)
    pe[:, 0::2] = np.sin(position * div_term)
    pe[:, 1::2] = np.cos(position * div_term)
    return pe


def _lane_offsets():
    off = np.zeros((16,), dtype=np.int32)
    for j in range(3):
        off[j] = j * _M2D
    for j in range(5):
        off[3 + j] = _NPOS + j * (2 * _M2D + 1) + _M2D
    off[8:] = off[:8] + _NBR
    return off


def _sc_body(t_hbm, f_hbm, pe_hbm, off_hbm, outv_hbm, outt_hbm,
             frow_v, idx_v, rows0, rows1, pe_v,
             outv0, outt0, outv1, outt1, off_v,
             sem_g0, sem_g1, sem_wb0, sem_wb1):
    wid = lax.axis_index("c") * 16 + lax.axis_index("s")
    s_base = wid * _SPW

    pltpu.sync_copy(off_hbm, off_v)
    pltpu.sync_copy(pe_hbm.at[pl.ds(s_base, _SPW)], pe_v)
    off = off_v[...]

    def idx_row(sl, carry):
        pltpu.sync_copy(f_hbm.at[s_base + sl, :], frow_v)
        for k in range(_B):
            fch = frow_v[pl.ds(k * 16, 16)]
            clipped = jnp.minimum(jnp.maximum(fch, -_M2D), _M2D)
            idx_v[sl * _BG + (k // _CH), pl.ds((k % _CH) * 16, 16)] = \
                clipped + off
        return carry

    lax.fori_loop(0, _SPW, idx_row, 0)

    def fire_gather(g, rows, sem):
        return pltpu.async_copy(t_hbm.at[idx_v.at[g]], rows, sem)

    def compute(g, rows, outv, outt):
        sl = g // _BG
        b0 = (g % _BG) * _CH
        s = s_base + sl
        for i in range(8):
            for c in range(6):
                col = c * 16
                seg = i * _SUB + col
                pe_c = pe_v[sl, pl.ds(seg, 16)]
                for t in range(_CH):
                    xr = t * 16 + i
                    yr = t * 16 + 8 + i
                    outv[t, pl.ds(seg, 16)] = \
                        rows[xr, pl.ds(col, 16)] + rows[yr, pl.ds(col, 16)] \
                        + pe_c
                    outt[t, pl.ds(seg, 16)] = \
                        rows[xr, pl.ds(_SUB + col, 16)] \
                        + rows[yr, pl.ds(_SUB + col, 16)] + pe_c
        pltpu.async_copy(outv, outv_hbm.at[pl.ds(b0, _CH), s, :],
                         sem_wb0 if (outv is outv0) else sem_wb1)
        pltpu.async_copy(outt, outt_hbm.at[pl.ds(b0, _CH), s, :],
                         sem_wb0 if (outv is outv0) else sem_wb1)

    def drain_wb(outv, outt, sem):
        pltpu.make_async_copy(outv, outv_hbm.at[pl.ds(0, _CH), 0, :], sem).wait()
        pltpu.make_async_copy(outt, outt_hbm.at[pl.ds(0, _CH), 0, :], sem).wait()

    fire_gather(0, rows0, sem_g0)

    def body(g2, carry):
        a = 2 * g2
        b = a + 1

        fire_gather(b, rows1, sem_g1)

        pltpu.make_async_copy(t_hbm.at[idx_v.at[a]], rows0, sem_g0).wait()

        @pl.when(g2 > 0)
        def _():
            drain_wb(outv0, outt0, sem_wb0)
        compute(a, rows0, outv0, outt0)

        @pl.when(g2 < _NCHUNK // 2 - 1)
        def _():
            fire_gather(a + 2, rows0, sem_g0)

        pltpu.make_async_copy(t_hbm.at[idx_v.at[b]], rows1, sem_g1).wait()

        @pl.when(g2 > 0)
        def _():
            drain_wb(outv1, outt1, sem_wb1)
        compute(b, rows1, outv1, outt1)
        return carry

    lax.fori_loop(0, _NCHUNK // 2, body, 0)

    drain_wb(outv0, outt0, sem_wb0)
    drain_wb(outv1, outt1, sem_wb1)


def kernel(x_feature, y_feature, pos_vx, dist_vx, pos_vy, dist_vy,
           pos_tx, dist_tx, pos_ty, dist_ty):
    tvx = jnp.concatenate([pos_vx.reshape(_NPOS, _SUB),
                           dist_vx.reshape(_NDIST, _SUB)], axis=0)
    ttx = jnp.concatenate([pos_tx.reshape(_NPOS, _SUB),
                           dist_tx.reshape(_NDIST, _SUB)], axis=0)
    tvy = jnp.concatenate([pos_vy.reshape(_NPOS, _SUB),
                           dist_vy.reshape(_NDIST, _SUB)], axis=0)
    tty = jnp.concatenate([pos_ty.reshape(_NPOS, _SUB),
                           dist_ty.reshape(_NDIST, _SUB)], axis=0)
    tx = jnp.concatenate([tvx, ttx], axis=1)
    ty = jnp.concatenate([tvy, tty], axis=1)
    table = jnp.concatenate([tx, ty], axis=0)

    feats = jnp.concatenate([x_feature, y_feature], axis=-1)
    f_sm = feats.reshape(_S, _B * 16)
    table = jnp.concatenate([tx, ty], axis=0)

    pe = jnp.asarray(_pe_table())
    off = jnp.asarray(_lane_offsets())

    run = pl.kernel(
        _sc_body,
        out_type=[
            jax.ShapeDtypeStruct((_B, _S, _H), jnp.float32),
            jax.ShapeDtypeStruct((_B, _S, _H), jnp.float32),
        ],
        mesh=plsc.VectorSubcoreMesh(core_axis_name="c", subcore_axis_name="s"),
        compiler_params=pltpu.CompilerParams(use_tc_tiling_on_sc=False),
        scratch_types=[
            pltpu.VMEM((_B * 16,), jnp.int32),
            pltpu.VMEM((_NCHUNK, _RPC), jnp.int32),
            pltpu.VMEM((_RPC, 2 * _SUB), jnp.float32),
            pltpu.VMEM((_RPC, 2 * _SUB), jnp.float32),
            pltpu.VMEM((_SPW, _H), jnp.float32),
            pltpu.VMEM((_CH, _H), jnp.float32),
            pltpu.VMEM((_CH, _H), jnp.float32),
            pltpu.VMEM((_CH, _H), jnp.float32),
            pltpu.VMEM((_CH, _H), jnp.float32),
            pltpu.VMEM((16,), jnp.int32),
            pltpu.SemaphoreType.DMA,
            pltpu.SemaphoreType.DMA,
            pltpu.SemaphoreType.DMA,
            pltpu.SemaphoreType.DMA,
        ],
    )
    outv, outt = run(table, f_sm, pe, off)
    return outv, outt


# speedup vs baseline: 1.4756x; 1.4756x over previous
import numpy as np
import jax
import jax.numpy as jnp
from jax import lax
from jax.experimental import pallas as pl
from jax.experimental.pallas import tpu as pltpu
from jax.experimental.pallas import tpu_sc as plsc

_H = 768
_M2D = 1024
_SUB = 96
_B = 128
_S = 512
_MAXP = 512

_NPOS = 3 * _M2D
_NDIST = 5 * (2 * _M2D + 1)
_NBR = _NPOS + _NDIST

_NW = 32
_SPW = _S // _NW
_CH = 8
_RPC = _CH * 16
_BG = _B // _CH
_NCHUNK = _SPW * _BG


def _pe_table():
    position = np.arange(_MAXP)[:, None].astype(np.float32)
    div_term = np.exp(
        np.arange(0, _H, 2).astype(np.float32) * (-np.log(10000.0) / _H))
    pe = np.zeros((_MAXP, _H), dtype=np.float32)
    pe[:, 0::2] = np.sin(position * div_term)
    pe[:, 1::2] = np.cos(position * div_term)
    return pe


def _lane_offsets():
    off = np.zeros((16,), dtype=np.int32)
    for j in range(3):
        off[j] = j * _M2D
    for j in range(5):
        off[3 + j] = _NPOS + j * (2 * _M2D + 1) + _M2D
    off[8:] = off[:8] + _NBR
    return off


def _sc_body(t_hbm, f_hbm, pe_hbm, off_hbm, outv_hbm, outt_hbm,
             frow_v, idx_v, rows0, rows1, pe_v,
             outv0, outt0, outv1, outt1, off_v,
             sem_g0, sem_g1, sem_wb0, sem_wb1):
    wid = lax.axis_index("c") * 16 + lax.axis_index("s")
    s_base = wid * _SPW

    pltpu.sync_copy(off_hbm, off_v)
    pltpu.sync_copy(pe_hbm.at[pl.ds(s_base, _SPW)], pe_v)
    off = off_v[...]

    def idx_row(sl, carry):
        pltpu.sync_copy(f_hbm.at[s_base + sl, :], frow_v)
        for k in range(_B):
            fch = frow_v[pl.ds(k * 16, 16)]
            clipped = jnp.minimum(jnp.maximum(fch, -_M2D), _M2D)
            idx_v[sl * _BG + (k // _CH), pl.ds((k % _CH) * 16, 16)] = \
                clipped + off
        return carry

    lax.fori_loop(0, _SPW, idx_row, 0)

    def fire_gather(g, rows, sem):
        return pltpu.async_copy(t_hbm.at[idx_v.at[g]], rows, sem)

    def compute(g, rows, outv, outt):
        sl = g // _BG
        b0 = (g % _BG) * _CH
        sblk = wid * 2 + sl // _CH
        r = sl % _CH
        for i in range(8):
            for c in range(6):
                col = c * 16
                seg = i * _SUB + col
                tj = seg // 128
                tcol = seg % 128
                pe_c = pe_v[sl, pl.ds(seg, 16)]
                for t in range(_CH):
                    xr = t * 16 + i
                    yr = t * 16 + 8 + i
                    outv[t, tj, pl.ds(tcol, 16)] = \
                        rows[xr, pl.ds(col, 16)] + rows[yr, pl.ds(col, 16)] \
                        + pe_c
                    outt[t, tj, pl.ds(tcol, 16)] = \
                        rows[xr, pl.ds(_SUB + col, 16)] \
                        + rows[yr, pl.ds(_SUB + col, 16)] + pe_c
        pltpu.async_copy(outv, outv_hbm.at[pl.ds(b0, _CH), sblk, :, r, :],
                         sem_wb0 if (outv is outv0) else sem_wb1)
        pltpu.async_copy(outt, outt_hbm.at[pl.ds(b0, _CH), sblk, :, r, :],
                         sem_wb0 if (outv is outv0) else sem_wb1)

    def drain_wb(outv, outt, sem):
        pltpu.make_async_copy(
            outv, outv_hbm.at[pl.ds(0, _CH), 0, :, 0, :], sem).wait()
        pltpu.make_async_copy(
            outt, outt_hbm.at[pl.ds(0, _CH), 0, :, 0, :], sem).wait()

    fire_gather(0, rows0, sem_g0)

    def body(g2, carry):
        a = 2 * g2
        b = a + 1

        fire_gather(b, rows1, sem_g1)

        pltpu.make_async_copy(t_hbm.at[idx_v.at[a]], rows0, sem_g0).wait()

        @pl.when(g2 > 0)
        def _():
            drain_wb(outv0, outt0, sem_wb0)
        compute(a, rows0, outv0, outt0)

        @pl.when(g2 < _NCHUNK // 2 - 1)
        def _():
            fire_gather(a + 2, rows0, sem_g0)

        pltpu.make_async_copy(t_hbm.at[idx_v.at[b]], rows1, sem_g1).wait()

        @pl.when(g2 > 0)
        def _():
            drain_wb(outv1, outt1, sem_wb1)
        compute(b, rows1, outv1, outt1)
        return carry

    lax.fori_loop(0, _NCHUNK // 2, body, 0)

    drain_wb(outv0, outt0, sem_wb0)
    drain_wb(outv1, outt1, sem_wb1)


def kernel(x_feature, y_feature, pos_vx, dist_vx, pos_vy, dist_vy,
           pos_tx, dist_tx, pos_ty, dist_ty):
    tvx = jnp.concatenate([pos_vx.reshape(_NPOS, _SUB),
                           dist_vx.reshape(_NDIST, _SUB)], axis=0)
    ttx = jnp.concatenate([pos_tx.reshape(_NPOS, _SUB),
                           dist_tx.reshape(_NDIST, _SUB)], axis=0)
    tvy = jnp.concatenate([pos_vy.reshape(_NPOS, _SUB),
                           dist_vy.reshape(_NDIST, _SUB)], axis=0)
    tty = jnp.concatenate([pos_ty.reshape(_NPOS, _SUB),
                           dist_ty.reshape(_NDIST, _SUB)], axis=0)
    tx = jnp.concatenate([tvx, ttx], axis=1)
    ty = jnp.concatenate([tvy, tty], axis=1)
    table = jnp.concatenate([tx, ty], axis=0)

    feats = jnp.concatenate([x_feature, y_feature], axis=-1)
    f_sm = feats.transpose(1, 0, 2).reshape(_S, _B * 16)

    pe = jnp.asarray(_pe_table())
    off = jnp.asarray(_lane_offsets())

    run = pl.kernel(
        _sc_body,
        out_type=[
            jax.ShapeDtypeStruct((_B, _S // 8, _H // 128, 8, 128), jnp.float32),
            jax.ShapeDtypeStruct((_B, _S // 8, _H // 128, 8, 128), jnp.float32),
        ],
        mesh=plsc.VectorSubcoreMesh(core_axis_name="c", subcore_axis_name="s"),
        compiler_params=pltpu.CompilerParams(use_tc_tiling_on_sc=False),
        scratch_types=[
            pltpu.VMEM((_B * 16,), jnp.int32),
            pltpu.VMEM((_NCHUNK, _RPC), jnp.int32),
            pltpu.VMEM((_RPC, 2 * _SUB), jnp.float32),
            pltpu.VMEM((_RPC, 2 * _SUB), jnp.float32),
            pltpu.VMEM((_SPW, _H), jnp.float32),
            pltpu.VMEM((_CH, _H // 128, 128), jnp.float32),
            pltpu.VMEM((_CH, _H // 128, 128), jnp.float32),
            pltpu.VMEM((_CH, _H // 128, 128), jnp.float32),
            pltpu.VMEM((_CH, _H // 128, 128), jnp.float32),
            pltpu.VMEM((16,), jnp.int32),
            pltpu.SemaphoreType.DMA,
            pltpu.SemaphoreType.DMA,
            pltpu.SemaphoreType.DMA,
            pltpu.SemaphoreType.DMA,
        ],
    )
    outv, outt = run(table, f_sm, pe, off)
    outv = jnp.transpose(outv, (0, 1, 3, 2, 4)).reshape(_B, _S, _H)
    outt = jnp.transpose(outt, (0, 1, 3, 2, 4)).reshape(_B, _S, _H)
    return outv, outt
